# Initial kernel scaffold; baseline (speedup 1.0000x reference)
import jax
import jax.numpy as jnp
from jax.experimental import pallas as pl


def kernel(x, edge_index, W_rel, b_rel, W_root, W1, att_src1, att_dst1, b1, W2, att_src2, att_dst2, b2, Wr, br):
    raise NotImplementedError("write your pallas kernel here")


# speedup vs baseline: 1.0530x; 1.0530x over previous
import functools

import jax
import jax.numpy as jnp
from jax import lax
from jax.experimental import pallas as pl
from jax.experimental.pallas import tpu as pltpu
from jax.experimental.pallas import tpu_sc as plsc

N = 10000
E = 320000
D = 128

NC = 2
NS = 16
NW = NC * NS

CHUNK = 128

EP1 = 327680
PER_SUB1 = EP1 // NW
NCHUNK1 = PER_SUB1 // CHUNK

NP = 10112
ROWS_PER_SUB = NP // NS


def _agg_body(src_hbm, dst_hbm, x_hbm, zeros_hbm, out_hbm, acc_sh, sidx_v,
              didx_v, buf_v, sem):
    cid = lax.axis_index("c")
    sid = lax.axis_index("s")

    lo = sid * ROWS_PER_SUB
    pltpu.sync_copy(zeros_hbm.at[pl.ds(lo, ROWS_PER_SUB)],
                    acc_sh.at[pl.ds(lo, ROWS_PER_SUB)])
    plsc.subcore_barrier()

    base = cid * (EP1 // NC) + sid * PER_SUB1

    def step(i, carry):
        off = base + i * CHUNK
        pltpu.sync_copy(src_hbm.at[pl.ds(off, CHUNK)], sidx_v)
        pltpu.sync_copy(dst_hbm.at[pl.ds(off, CHUNK)], didx_v)
        pltpu.async_copy(x_hbm.at[sidx_v], buf_v, sem).wait()
        pltpu.sync_copy(buf_v, acc_sh.at[didx_v], add=True)
        return carry

    lax.fori_loop(0, NCHUNK1, step, 0)
    plsc.subcore_barrier()
    pltpu.sync_copy(acc_sh.at[pl.ds(lo, ROWS_PER_SUB)],
                    out_hbm.at[cid, pl.ds(lo, ROWS_PER_SUB)])


@jax.jit
def _sc_agg(srcp, dstp, x, zeros):
    mesh = plsc.VectorSubcoreMesh(core_axis_name="c", subcore_axis_name="s")
    return pl.kernel(
        _agg_body,
        out_type=jax.ShapeDtypeStruct((NC, NP, D), jnp.float32),
        mesh=mesh,
        scratch_types=[
            pltpu.VMEM_SHARED((NP, D), jnp.float32),
            pltpu.VMEM((CHUNK,), jnp.int32),
            pltpu.VMEM((CHUNK,), jnp.int32),
            pltpu.VMEM((CHUNK, D), jnp.float32),
            pltpu.SemaphoreType.DMA,
        ],
    )(srcp, dstp, x, zeros)


def _gat_ref(h, src, dst, W, a_src, a_dst, b, H, C, concat):
    xW = (h @ W).reshape(-1, H, C)
    es = jnp.sum(xW * a_src, axis=-1)
    ed = jnp.sum(xW * a_dst, axis=-1)
    e = jax.nn.leaky_relu(es[src] + ed[dst], negative_slope=0.2)
    m = jax.ops.segment_max(e, dst, num_segments=N)
    ex = jnp.exp(e - m[dst])
    den = jax.ops.segment_sum(ex, dst, num_segments=N)
    alpha = ex / (den[dst] + 1e-16)
    out = jax.ops.segment_sum(xW[src] * alpha[:, :, None], dst, num_segments=N)
    if concat:
        out = out.reshape(-1, H * C)
    else:
        out = out.mean(axis=1)
    return out + b


def kernel(x, edge_index, W_rel, b_rel, W_root, W1, att_src1, att_dst1, b1,
           W2, att_src2, att_dst2, b2, Wr, br):
    src, dst = edge_index[0], edge_index[1]

    pad1 = EP1 - E
    srcp = jnp.concatenate([src, (jnp.arange(pad1, dtype=jnp.int32) % N)])
    dstp = jnp.concatenate([dst, N + (jnp.arange(pad1, dtype=jnp.int32) % 32)])
    zeros = jnp.zeros((NP, D), jnp.float32)

    aggs = _sc_agg(srcp, dstp, x, zeros)
    agg = aggs[0, :N] + aggs[1, :N]

    h = agg @ W_rel + b_rel + x @ W_root
    h = jax.nn.elu(h)

    loop = jnp.arange(N, dtype=edge_index.dtype)
    src2 = jnp.concatenate([src, loop])
    dst2 = jnp.concatenate([dst, loop])
    h = _gat_ref(h, src2, dst2, W1, att_src1, att_dst1, b1, 8, 8, True)
    h = jax.nn.elu(h)
    h = _gat_ref(h, src2, dst2, W2, att_src2, att_dst2, b2, 1, 64, False)
    g = jnp.mean(h, axis=0, keepdims=True)
    return g @ Wr + br


# speedup vs baseline: 29.8021x; 28.3017x over previous
import functools

import jax
import jax.numpy as jnp
from jax import lax
from jax.experimental import pallas as pl
from jax.experimental.pallas import tpu as pltpu
from jax.experimental.pallas import tpu_sc as plsc

N = 10000
E = 320000
D = 128

NC = 2
NS = 16
NW = NC * NS

CHUNK = 128

EP1 = 327680
PER_SUB1 = EP1 // NW
NCHUNK1 = PER_SUB1 // CHUNK

NP = 10112
ROWS_PER_SUB = NP // NS


def _agg_body(src_hbm, dst_hbm, x_hbm, zeros_hbm, out_hbm, acc_sh, sidx_v,
              didx_v, buf_v, sem):
    cid = lax.axis_index("c")
    sid = lax.axis_index("s")

    lo = sid * ROWS_PER_SUB
    pltpu.sync_copy(zeros_hbm.at[pl.ds(lo, ROWS_PER_SUB)],
                    acc_sh.at[pl.ds(lo, ROWS_PER_SUB)])
    plsc.subcore_barrier()

    base = cid * (EP1 // NC) + sid * PER_SUB1

    def step(i, carry):
        off = base + i * CHUNK
        pltpu.sync_copy(src_hbm.at[pl.ds(off, CHUNK)], sidx_v)
        pltpu.sync_copy(dst_hbm.at[pl.ds(off, CHUNK)], didx_v)
        pltpu.async_copy(x_hbm.at[sidx_v], buf_v, sem).wait()
        pltpu.sync_copy(buf_v, acc_sh.at[didx_v], add=True)
        return carry

    lax.fori_loop(0, NCHUNK1, step, 0)
    plsc.subcore_barrier()
    pltpu.sync_copy(acc_sh.at[pl.ds(lo, ROWS_PER_SUB)],
                    out_hbm.at[cid, pl.ds(lo, ROWS_PER_SUB)])


@jax.jit
def _sc_agg(srcp, dstp, x, zeros):
    mesh = plsc.VectorSubcoreMesh(core_axis_name="c", subcore_axis_name="s")
    return pl.kernel(
        _agg_body,
        out_type=jax.ShapeDtypeStruct((NC, NP, D), jnp.float32),
        mesh=mesh,
        compiler_params=pltpu.CompilerParams(needs_layout_passes=False),
        scratch_types=[
            pltpu.VMEM_SHARED((NP, D), jnp.float32),
            pltpu.VMEM((CHUNK,), jnp.int32),
            pltpu.VMEM((CHUNK,), jnp.int32),
            pltpu.VMEM((CHUNK, D), jnp.float32),
            pltpu.SemaphoreType.DMA,
        ],
    )(srcp, dstp, x, zeros)


CH2 = 96
EP2 = 331776
PER_SUB2 = EP2 // NW
NCHUNK2 = PER_SUB2 // CH2

F = 64


def _gat_body(src_hbm, dst_hbm, tbl_hbm, zeros_hbm, acc_out,
              acc_sh, sidx_v, didx_v, sbuf, dbuf, wout, exf, sem):
    cid = lax.axis_index("c")
    sid = lax.axis_index("s")

    lo = sid * ROWS_PER_SUB
    pltpu.sync_copy(zeros_hbm.at[pl.ds(lo, ROWS_PER_SUB)],
                    acc_sh.at[pl.ds(lo, ROWS_PER_SUB)])
    pltpu.sync_copy(zeros_hbm.at[pl.ds(0, CH2)], wout)
    plsc.subcore_barrier()

    pat8 = lax.div(lax.iota(jnp.int32, 16), 8)

    base = cid * (EP2 // NC) + sid * PER_SUB2

    def step(i, carry):
        off = base + i * CH2
        pltpu.sync_copy(src_hbm.at[pl.ds(off, CH2)], sidx_v)
        pltpu.sync_copy(dst_hbm.at[pl.ds(off, CH2)], didx_v)
        cp1 = pltpu.async_copy(tbl_hbm.at[sidx_v], sbuf, sem)
        cp2 = pltpu.async_copy(tbl_hbm.at[didx_v], dbuf, sem)
        cp1.wait()
        cp2.wait()

        def exstep(e, c):
            z = sbuf[e, pl.ds(64, 16)] + dbuf[e, pl.ds(80, 16)]
            z = jnp.maximum(z, 0.2 * z)
            v = jnp.exp(jnp.maximum(z - dbuf[e, pl.ds(96, 16)], -80.0))
            wout[e, pl.ds(64, 16)] = v
            exf[pl.ds(e * 16, 16)] = v
            return c

        lax.fori_loop(0, CH2, exstep, 0)

        def wstep(j, c):
            e = lax.div(j, 4)
            q = lax.rem(j, 4)
            w = plsc.load_gather(exf, [e * 16 + 2 * q + pat8])
            wout[e, pl.ds(q * 16, 16)] = sbuf[e, pl.ds(q * 16, 16)] * w
            return c

        lax.fori_loop(0, CH2 * 4, wstep, 0)

        pltpu.sync_copy(wout, acc_sh.at[didx_v], add=True)
        return carry

    lax.fori_loop(0, NCHUNK2, step, 0)
    plsc.subcore_barrier()
    pltpu.sync_copy(acc_sh.at[pl.ds(lo, ROWS_PER_SUB)],
                    acc_out.at[cid, pl.ds(lo, ROWS_PER_SUB)])


@jax.jit
def _sc_gat(srcp, dstp, tbl, zeros):
    mesh = plsc.VectorSubcoreMesh(core_axis_name="c", subcore_axis_name="s")
    return pl.kernel(
        _gat_body,
        out_type=jax.ShapeDtypeStruct((NC, NP, D), jnp.float32),
        mesh=mesh,
        compiler_params=pltpu.CompilerParams(needs_layout_passes=False),
        scratch_types=[
            pltpu.VMEM_SHARED((NP, D), jnp.float32),
            pltpu.VMEM((CH2,), jnp.int32),
            pltpu.VMEM((CH2,), jnp.int32),
            pltpu.VMEM((CH2, D), jnp.float32),
            pltpu.VMEM((CH2, D), jnp.float32),
            pltpu.VMEM((CH2, D), jnp.float32),
            pltpu.VMEM((CH2 * 16,), jnp.float32),
            pltpu.SemaphoreType.DMA,
        ],
    )(srcp, dstp, tbl, zeros)


def kernel(x, edge_index, W_rel, b_rel, W_root, W1, att_src1, att_dst1, b1,
           W2, att_src2, att_dst2, b2, Wr, br):
    src, dst = edge_index[0], edge_index[1]

    pad1 = EP1 - E
    srcp = jnp.concatenate([src, (jnp.arange(pad1, dtype=jnp.int32) % N)])
    dstp = jnp.concatenate([dst, N + (jnp.arange(pad1, dtype=jnp.int32) % 32)])
    zeros = jnp.zeros((NP, D), jnp.float32)

    aggs = _sc_agg(srcp, dstp, x, zeros)
    agg = aggs[0, :N] + aggs[1, :N]

    h = agg @ W_rel + b_rel + x @ W_root
    h = jax.nn.elu(h)

    loop = jnp.arange(N, dtype=jnp.int32)
    pad2 = EP2 - E - N
    src2 = jnp.concatenate([src, loop, (jnp.arange(pad2, dtype=jnp.int32) % N)])
    dst2 = jnp.concatenate([dst, loop,
                            N + (jnp.arange(pad2, dtype=jnp.int32) % 32)])

    xw1 = h @ W1
    xw1r = xw1.reshape(N, 8, 8)
    es1 = jnp.sum(xw1r * att_src1, axis=-1)
    ed1 = jnp.sum(xw1r * att_dst1, axis=-1)
    md1 = jnp.max(es1, axis=0) + ed1
    md1 = jnp.maximum(md1, 0.2 * md1)
    tbl1 = jnp.concatenate(
        [xw1, es1, es1, ed1, ed1, md1, md1, jnp.zeros((N, 16), jnp.float32)],
        axis=1)
    acc1 = _sc_gat(src2, dst2, tbl1, zeros)
    accs1 = acc1[0, :N] + acc1[1, :N]
    num = accs1[:, :64].reshape(N, 8, 8)
    den = accs1[:, 64:72]
    h1 = (num / (den[:, :, None] + 1e-16)).reshape(N, 64) + b1
    h1 = jax.nn.elu(h1)

    xw2 = h1 @ W2
    es2 = jnp.sum(xw2 * att_src2[0, 0], axis=-1)
    ed2 = jnp.sum(xw2 * att_dst2[0, 0], axis=-1)
    md2 = jnp.max(es2) + ed2
    md2 = jnp.maximum(md2, 0.2 * md2)
    ones16 = jnp.ones((1, 16), jnp.float32)
    tbl2 = jnp.concatenate(
        [xw2, es2[:, None] * ones16, ed2[:, None] * ones16,
         md2[:, None] * ones16, jnp.zeros((N, 16), jnp.float32)], axis=1)
    acc2 = _sc_gat(src2, dst2, tbl2, zeros)
    accs2 = acc2[0, :N] + acc2[1, :N]
    h2 = accs2[:, :64] / (accs2[:, 64:65] + 1e-16) + b2

    g = jnp.mean(h2, axis=0, keepdims=True)
    return g @ Wr + br


# speedup vs baseline: 30.3196x; 1.0174x over previous
import functools

import jax
import jax.numpy as jnp
from jax import lax
from jax.experimental import pallas as pl
from jax.experimental.pallas import tpu as pltpu
from jax.experimental.pallas import tpu_sc as plsc

N = 10000
E = 320000
D = 128

NC = 2
NS = 16
NW = NC * NS

CHUNK = 128

EP1 = 327680
PER_SUB1 = EP1 // NW
NCHUNK1 = PER_SUB1 // CHUNK

NP = 10112
ROWS_PER_SUB = NP // NS


def _agg_body(src_hbm, dst_hbm, x_hbm, zeros_hbm, out_hbm, acc_sh, sidx_v,
              didx_v, buf_v, sem):
    cid = lax.axis_index("c")
    sid = lax.axis_index("s")

    lo = sid * ROWS_PER_SUB
    pltpu.sync_copy(zeros_hbm.at[pl.ds(lo, ROWS_PER_SUB)],
                    acc_sh.at[pl.ds(lo, ROWS_PER_SUB)])
    plsc.subcore_barrier()

    base = cid * (EP1 // NC) + sid * PER_SUB1

    def step(i, carry):
        off = base + i * CHUNK
        pltpu.sync_copy(src_hbm.at[pl.ds(off, CHUNK)], sidx_v)
        pltpu.sync_copy(dst_hbm.at[pl.ds(off, CHUNK)], didx_v)
        pltpu.async_copy(x_hbm.at[sidx_v], buf_v, sem).wait()
        pltpu.sync_copy(buf_v, acc_sh.at[didx_v], add=True)
        return carry

    lax.fori_loop(0, NCHUNK1, step, 0)
    plsc.subcore_barrier()
    pltpu.sync_copy(acc_sh.at[pl.ds(lo, ROWS_PER_SUB)],
                    out_hbm.at[cid, pl.ds(lo, ROWS_PER_SUB)])


@jax.jit
def _sc_agg(srcp, dstp, x, zeros):
    mesh = plsc.VectorSubcoreMesh(core_axis_name="c", subcore_axis_name="s")
    return pl.kernel(
        _agg_body,
        out_type=jax.ShapeDtypeStruct((NC, NP, D), jnp.float32),
        mesh=mesh,
        compiler_params=pltpu.CompilerParams(needs_layout_passes=False),
        scratch_types=[
            pltpu.VMEM_SHARED((NP, D), jnp.float32),
            pltpu.VMEM((CHUNK,), jnp.int32),
            pltpu.VMEM((CHUNK,), jnp.int32),
            pltpu.VMEM((CHUNK, D), jnp.float32),
            pltpu.SemaphoreType.DMA,
        ],
    )(srcp, dstp, x, zeros)


CH2 = 96
EP2 = 331776
PER_SUB2 = EP2 // NW
NCHUNK2 = PER_SUB2 // CH2

F = 64


def _gat_body(src_hbm, dst_hbm, tbl_hbm, mv_hbm, zeros_hbm, acc_out,
              acc_sh, sidx_v, didx_v, sbuf, dbuf, wout, exf, mv_v, sem):
    cid = lax.axis_index("c")
    sid = lax.axis_index("s")

    lo = sid * ROWS_PER_SUB
    pltpu.sync_copy(zeros_hbm.at[pl.ds(lo, ROWS_PER_SUB)],
                    acc_sh.at[pl.ds(lo, ROWS_PER_SUB)])
    pltpu.sync_copy(zeros_hbm.at[pl.ds(0, CH2)], wout)
    pltpu.sync_copy(mv_hbm, mv_v)
    plsc.subcore_barrier()

    mx = mv_v[...]
    pat8 = lax.div(lax.iota(jnp.int32, 16), 8)

    base = cid * (EP2 // NC) + sid * PER_SUB2

    def step(i, carry):
        off = base + i * CH2
        pltpu.sync_copy(src_hbm.at[pl.ds(off, CH2)], sidx_v)
        pltpu.sync_copy(dst_hbm.at[pl.ds(off, CH2)], didx_v)
        cp1 = pltpu.async_copy(tbl_hbm.at[sidx_v], sbuf, sem)
        cp2 = pltpu.async_copy(tbl_hbm.at[didx_v], dbuf, sem)
        cp1.wait()
        cp2.wait()

        def exstep(e, c):
            ed_v = dbuf[e, pl.ds(80, 16)]
            z = sbuf[e, pl.ds(64, 16)] + ed_v
            z = jnp.maximum(z, 0.2 * z)
            m = mx + ed_v
            m = jnp.maximum(m, 0.2 * m)
            v = jnp.exp(jnp.maximum(z - m, -80.0))
            wout[e, pl.ds(64, 16)] = v
            exf[pl.ds(e * 16, 16)] = v
            return c

        lax.fori_loop(0, CH2, exstep, 0)

        def wstep(j, c):
            e = lax.div(j, 4)
            q = lax.rem(j, 4)
            w = plsc.load_gather(exf, [e * 16 + 2 * q + pat8])
            wout[e, pl.ds(q * 16, 16)] = sbuf[e, pl.ds(q * 16, 16)] * w
            return c

        lax.fori_loop(0, CH2 * 4, wstep, 0)

        pltpu.sync_copy(wout, acc_sh.at[didx_v], add=True)
        return carry

    lax.fori_loop(0, NCHUNK2, step, 0)
    plsc.subcore_barrier()
    pltpu.sync_copy(acc_sh.at[pl.ds(lo, ROWS_PER_SUB)],
                    acc_out.at[cid, pl.ds(lo, ROWS_PER_SUB)])


@jax.jit
def _sc_gat(srcp, dstp, tbl, mv, zeros):
    mesh = plsc.VectorSubcoreMesh(core_axis_name="c", subcore_axis_name="s")
    return pl.kernel(
        _gat_body,
        out_type=jax.ShapeDtypeStruct((NC, NP, D), jnp.float32),
        mesh=mesh,
        compiler_params=pltpu.CompilerParams(needs_layout_passes=False),
        scratch_types=[
            pltpu.VMEM_SHARED((NP, D), jnp.float32),
            pltpu.VMEM((CH2,), jnp.int32),
            pltpu.VMEM((CH2,), jnp.int32),
            pltpu.VMEM((CH2, D), jnp.float32),
            pltpu.VMEM((CH2, D), jnp.float32),
            pltpu.VMEM((CH2, D), jnp.float32),
            pltpu.VMEM((CH2 * 16,), jnp.float32),
            pltpu.VMEM((16,), jnp.float32),
            pltpu.SemaphoreType.DMA,
        ],
    )(srcp, dstp, tbl, mv, zeros)


RB = 2000
GRID_N = N // RB


def _mm(a, b):
    return lax.dot_general(a, b, (((1,), (0,)), ((), ())),
                           precision=lax.Precision.HIGHEST,
                           preferred_element_type=jnp.float32)


def _tcA_body(agg0, agg1, x, Wrel, brel, Wroot, W1, As16, Ad16,
              xw_o, esx_o, edx_o, mx_o):
    h = agg0[...] + agg1[...]
    h = _mm(h, Wrel[...]) + brel[...] + _mm(x[...], Wroot[...])
    h = jnp.where(h > 0, h, jnp.exp(h) - 1.0)
    xw = _mm(h, W1[...])
    esx = _mm(xw, As16[...])
    edx = _mm(xw, Ad16[...])
    xw_o[...] = xw
    esx_o[...] = esx
    edx_o[...] = edx
    bm = jnp.max(esx, axis=0, keepdims=True)

    @pl.when(pl.program_id(0) == 0)
    def _():
        mx_o[...] = bm

    @pl.when(pl.program_id(0) != 0)
    def _():
        mx_o[...] = jnp.maximum(mx_o[...], bm)


@jax.jit
def _tc_stage1(agg0, agg1, x, Wrel, brel, Wroot, W1, As16, Ad16):
    row = lambda i: (i, 0)
    full = lambda i: (0, 0)
    return pl.pallas_call(
        _tcA_body,
        grid=(GRID_N,),
        in_specs=[
            pl.BlockSpec((RB, D), row), pl.BlockSpec((RB, D), row),
            pl.BlockSpec((RB, D), row),
            pl.BlockSpec((D, D), full), pl.BlockSpec((1, D), full),
            pl.BlockSpec((D, D), full), pl.BlockSpec((D, F), full),
            pl.BlockSpec((F, 16), full), pl.BlockSpec((F, 16), full),
        ],
        out_specs=[
            pl.BlockSpec((RB, F), row), pl.BlockSpec((RB, 16), row),
            pl.BlockSpec((RB, 16), row), pl.BlockSpec((1, 16), full),
        ],
        out_shape=[
            jax.ShapeDtypeStruct((N, F), jnp.float32),
            jax.ShapeDtypeStruct((N, 16), jnp.float32),
            jax.ShapeDtypeStruct((N, 16), jnp.float32),
            jax.ShapeDtypeStruct((1, 16), jnp.float32),
        ],
    )(agg0, agg1, x, Wrel, brel, Wroot, W1, As16, Ad16)


def _tcC_body(a0, a1, b1, W2, As2, Ad2, Exp8, xw_o, esx_o, edx_o, mx_o):
    s = a0[...] + a1[...]
    num = s[:, :64]
    den64 = _mm(s[:, 64:72], Exp8[...])
    h1 = num / (den64 + 1e-16) + b1[...]
    h1 = jnp.where(h1 > 0, h1, jnp.exp(h1) - 1.0)
    xw2 = _mm(h1, W2[...])
    esx = _mm(xw2, As2[...])
    edx = _mm(xw2, Ad2[...])
    xw_o[...] = xw2
    esx_o[...] = esx
    edx_o[...] = edx
    bm = jnp.max(esx, axis=0, keepdims=True)

    @pl.when(pl.program_id(0) == 0)
    def _():
        mx_o[...] = bm

    @pl.when(pl.program_id(0) != 0)
    def _():
        mx_o[...] = jnp.maximum(mx_o[...], bm)


@jax.jit
def _tc_stage2(a0, a1, b1, W2, As2, Ad2, Exp8):
    row = lambda i: (i, 0)
    full = lambda i: (0, 0)
    return pl.pallas_call(
        _tcC_body,
        grid=(GRID_N,),
        in_specs=[
            pl.BlockSpec((RB, D), row), pl.BlockSpec((RB, D), row),
            pl.BlockSpec((1, F), full), pl.BlockSpec((F, F), full),
            pl.BlockSpec((F, 16), full), pl.BlockSpec((F, 16), full),
            pl.BlockSpec((8, F), full),
        ],
        out_specs=[
            pl.BlockSpec((RB, F), row), pl.BlockSpec((RB, 16), row),
            pl.BlockSpec((RB, 16), row), pl.BlockSpec((1, 16), full),
        ],
        out_shape=[
            jax.ShapeDtypeStruct((N, F), jnp.float32),
            jax.ShapeDtypeStruct((N, 16), jnp.float32),
            jax.ShapeDtypeStruct((N, 16), jnp.float32),
            jax.ShapeDtypeStruct((1, 16), jnp.float32),
        ],
    )(a0, a1, b1, W2, As2, Ad2, Exp8)


def _tcE_body(a0, a1, b2, OnesE, Wr, br, out_o, acc):
    s = a0[...] + a1[...]
    num = s[:, :64]
    den64 = _mm(s[:, 64:65], OnesE[...])
    h2 = num / (den64 + 1e-16) + b2[...]
    part = jnp.sum(h2, axis=0, keepdims=True)

    @pl.when(pl.program_id(0) == 0)
    def _():
        acc[...] = part

    @pl.when(pl.program_id(0) != 0)
    def _():
        acc[...] = acc[...] + part

    @pl.when(pl.program_id(0) == GRID_N - 1)
    def _():
        g = acc[...] * (1.0 / N)
        out_o[...] = _mm(g, Wr[...]) + br[...]


@jax.jit
def _tc_stage3(a0, a1, b2, OnesE, Wr, br):
    row = lambda i: (i, 0)
    full = lambda i: (0, 0)
    return pl.pallas_call(
        _tcE_body,
        grid=(GRID_N,),
        in_specs=[
            pl.BlockSpec((RB, D), row), pl.BlockSpec((RB, D), row),
            pl.BlockSpec((1, F), full), pl.BlockSpec((1, F), full),
            pl.BlockSpec((F, 1), full), pl.BlockSpec((1, 1), full),
        ],
        out_specs=pl.BlockSpec((1, 1), full),
        out_shape=jax.ShapeDtypeStruct((1, 1), jnp.float32),
        scratch_shapes=[pltpu.VMEM((1, F), jnp.float32)],
    )(a0, a1, b2, OnesE, Wr, br)


def kernel(x, edge_index, W_rel, b_rel, W_root, W1, att_src1, att_dst1, b1,
           W2, att_src2, att_dst2, b2, Wr, br):
    src, dst = edge_index[0], edge_index[1]

    pad1 = EP1 - E
    srcp = jnp.concatenate([src, (jnp.arange(pad1, dtype=jnp.int32) % N)])
    dstp = jnp.concatenate([dst, N + (jnp.arange(pad1, dtype=jnp.int32) % 32)])
    zeros = jnp.zeros((NP, D), jnp.float32)

    aggs = _sc_agg(srcp, dstp, x, zeros)

    eye8 = jnp.eye(8, dtype=jnp.float32)
    As = (att_src1[0][:, :, None] * eye8[:, None, :]).reshape(64, 8)
    Ad = (att_dst1[0][:, :, None] * eye8[:, None, :]).reshape(64, 8)
    As16 = jnp.concatenate([As, As], axis=1)
    Ad16 = jnp.concatenate([Ad, Ad], axis=1)
    As2 = att_src2[0, 0][:, None] * jnp.ones((1, 16), jnp.float32)
    Ad2 = att_dst2[0, 0][:, None] * jnp.ones((1, 16), jnp.float32)
    Exp8 = jnp.repeat(jnp.eye(8, dtype=jnp.float32), 8, axis=1)
    OnesE = jnp.ones((1, F), jnp.float32)

    xw1, esx1, edx1, mx1 = _tc_stage1(
        aggs[0, :N], aggs[1, :N], x, W_rel, b_rel.reshape(1, D), W_root, W1,
        As16, Ad16)

    loop = jnp.arange(N, dtype=jnp.int32)
    pad2 = EP2 - E - N
    src2 = jnp.concatenate([src, loop, (jnp.arange(pad2, dtype=jnp.int32) % N)])
    dst2 = jnp.concatenate([dst, loop,
                            N + (jnp.arange(pad2, dtype=jnp.int32) % 32)])

    ztail = jnp.zeros((N, 32), jnp.float32)
    tbl1 = jnp.concatenate([xw1, esx1, edx1, ztail], axis=1)
    acc1 = _sc_gat(src2, dst2, tbl1, mx1.reshape(16), zeros)

    xw2, esx2, edx2, mx2 = _tc_stage2(
        acc1[0, :N], acc1[1, :N], b1.reshape(1, F), W2, As2, Ad2, Exp8)

    tbl2 = jnp.concatenate([xw2, esx2, edx2, ztail], axis=1)
    acc2 = _sc_gat(src2, dst2, tbl2, mx2.reshape(16), zeros)

    return _tc_stage3(acc2[0, :N], acc2[1, :N], b2.reshape(1, F), OnesE,
                      Wr.reshape(F, 1), br.reshape(1, 1))
